# Initial kernel scaffold; baseline (speedup 1.0000x reference)
import jax
import jax.numpy as jnp
from jax.experimental import pallas as pl


def kernel(des, tweet, num_prop, cat_prop, edge_index, edge_type, W_des, b_des, W_tw, b_tw, W_np, b_np, W_cp, b_cp, W_in, b_in, W_rel, W_root, b_rgcn, W_o1, b_o1, W_o2, b_o2):
    raise NotImplementedError("write your pallas kernel here")


# speedup vs baseline: 12.6468x; 12.6468x over previous
import functools

import jax
import jax.numpy as jnp
from jax import lax
from jax.experimental import pallas as pl
from jax.experimental.pallas import tpu as pltpu
from jax.experimental.pallas import tpu_sc as plsc

_N = 10000
_R = 5
_D = 128
_L = 16
_NC = 2
_NS = 16
_K = 80
_CNT_PAD = 65536
_SLICE = _CNT_PAD // _NS
_NPAD = 10112
_RPT = _NPAD // _NS
_BN = 1000

_HIGH = lax.Precision.HIGHEST


def _lrelu(v):
    return jnp.where(v >= 0, v, 0.01 * v)


def _dot(a, b):
    return jnp.dot(a, b, preferred_element_type=jnp.float32, precision=_HIGH)


def _encoder(des, tweet, npc, W_des, W_tw, W_npc, b_pre, W_in, b_in):
    n = des.shape[0]
    nb = n // _BN

    def body(des_r, tw_r, npc_r, wd_r, wt_r, wn_r, bp_r, wi_r, bi_r, o_r):
        d = _dot(des_r[...], wd_r[...])
        t = _dot(tw_r[...], wt_r[...])
        nc = _dot(npc_r[...], wn_r[...])
        xp = _lrelu(jnp.concatenate([d, t, nc], axis=1) + bp_r[...])
        o_r[...] = _lrelu(_dot(xp, wi_r[...]) + bi_r[...])

    return pl.pallas_call(
        body,
        grid=(nb,),
        in_specs=[
            pl.BlockSpec((_BN, 768), lambda i: (i, 0)),
            pl.BlockSpec((_BN, 768), lambda i: (i, 0)),
            pl.BlockSpec((_BN, 128), lambda i: (i, 0)),
            pl.BlockSpec((768, 32), lambda i: (0, 0)),
            pl.BlockSpec((768, 32), lambda i: (0, 0)),
            pl.BlockSpec((128, 64), lambda i: (0, 0)),
            pl.BlockSpec((1, 128), lambda i: (0, 0)),
            pl.BlockSpec((128, 128), lambda i: (0, 0)),
            pl.BlockSpec((1, 128), lambda i: (0, 0)),
        ],
        out_specs=pl.BlockSpec((_BN, 128), lambda i: (i, 0)),
        out_shape=jax.ShapeDtypeStruct((n, 128), jnp.float32),
    )(des, tweet, npc, W_des, W_tw, W_npc, b_pre, W_in, b_in)


def _relmm(x, W_rel):
    n = x.shape[0]
    nb = n // _BN

    def body(x_r, w_r, y_r):
        y_r[...] = _dot(x_r[...], w_r[0])

    return pl.pallas_call(
        body,
        grid=(_R, nb),
        in_specs=[
            pl.BlockSpec((_BN, 128), lambda r, i: (i, 0)),
            pl.BlockSpec((1, 128, 128), lambda r, i: (r, 0, 0)),
        ],
        out_specs=pl.BlockSpec((_BN, 128), lambda r, i: (r * nb + i, 0)),
        out_shape=jax.ShapeDtypeStruct((_R * n, 128), jnp.float32),
    )(x, W_rel)


def _inv(cnts):
    rows = cnts.shape[1]

    def body(c_r, o_r):
        o_r[...] = 1.0 / jnp.maximum(c_r[0] + c_r[1], 1.0)

    return pl.pallas_call(
        body,
        in_specs=[pl.BlockSpec((2, rows, 128), lambda: (0, 0, 0))],
        out_specs=pl.BlockSpec((rows, 128), lambda: (0, 0)),
        out_shape=jax.ShapeDtypeStruct((rows, 128), jnp.float32),
    )(cnts)


def _combine(x, p0, p1, W_root, b):
    n = x.shape[0]
    nb = n // _BN

    def body(x_r, p0_r, p1_r, w_r, b_r, o_r):
        o_r[...] = _dot(x_r[...], w_r[...]) + b_r[...] + p0_r[...] + p1_r[...]

    return pl.pallas_call(
        body,
        grid=(nb,),
        in_specs=[
            pl.BlockSpec((_BN, 128), lambda i: (i, 0)),
            pl.BlockSpec((_BN, 128), lambda i: (i, 0)),
            pl.BlockSpec((_BN, 128), lambda i: (i, 0)),
            pl.BlockSpec((128, 128), lambda i: (0, 0)),
            pl.BlockSpec((1, 128), lambda i: (0, 0)),
        ],
        out_specs=pl.BlockSpec((_BN, 128), lambda i: (i, 0)),
        out_shape=jax.ShapeDtypeStruct((n, 128), jnp.float32),
    )(x, p0, p1, W_root, b)


def _final(x, p0, p1, W_root, b_rgcn, W_o1, b_o1, W_o2p, b_o2p):
    n = x.shape[0]
    nb = n // _BN

    def body(x_r, p0_r, p1_r, wr_r, br_r, w1_r, b1_r, w2_r, b2_r, o_r):
        x3 = _dot(x_r[...], wr_r[...]) + br_r[...] + p0_r[...] + p1_r[...]
        h = _lrelu(_dot(x3, w1_r[...]) + b1_r[...])
        o_r[...] = _dot(h, w2_r[...]) + b2_r[...]

    return pl.pallas_call(
        body,
        grid=(nb,),
        in_specs=[
            pl.BlockSpec((_BN, 128), lambda i: (i, 0)),
            pl.BlockSpec((_BN, 128), lambda i: (i, 0)),
            pl.BlockSpec((_BN, 128), lambda i: (i, 0)),
            pl.BlockSpec((128, 128), lambda i: (0, 0)),
            pl.BlockSpec((1, 128), lambda i: (0, 0)),
            pl.BlockSpec((128, 128), lambda i: (0, 0)),
            pl.BlockSpec((1, 128), lambda i: (0, 0)),
            pl.BlockSpec((128, 128), lambda i: (0, 0)),
            pl.BlockSpec((1, 128), lambda i: (0, 0)),
        ],
        out_specs=pl.BlockSpec((_BN, 128), lambda i: (i, 0)),
        out_shape=jax.ShapeDtypeStruct((n, 128), jnp.float32),
    )(x, p0, p1, W_root, b_rgcn, W_o1, b_o1, W_o2p, b_o2p)


_MESH = dict(core_axis_name="c", subcore_axis_name="s")


def _sc_count(dst1, typ1):
    e_per = dst1.shape[0] // (_NC * _NS)
    n_ch = e_per // _K

    @functools.partial(
        pl.kernel,
        out_type=jax.ShapeDtypeStruct((_NC * _CNT_PAD,), jnp.float32),
        mesh=plsc.VectorSubcoreMesh(**_MESH),
        scratch_types=[
            pltpu.VMEM((e_per,), jnp.int32),
            pltpu.VMEM((e_per,), jnp.int32),
            pltpu.VMEM((_K,), jnp.float32),
            pltpu.VMEM((_SLICE,), jnp.float32),
            pltpu.VMEM((1, _K), jnp.int32),
            pltpu.VMEM_SHARED((_CNT_PAD,), jnp.float32),
        ],
    )
    def run(dst_h, typ_h, out_h, dbuf, tbuf, ones, zbuf, ibuf, cnt_sh):
        c = lax.axis_index("c")
        s = lax.axis_index("s")

        @pl.loop(0, _SLICE // _L)
        def _(i):
            zbuf[pl.ds(i * _L, _L)] = jnp.zeros((_L,), jnp.float32)

        @pl.loop(0, _K // _L)
        def _(i):
            ones[pl.ds(i * _L, _L)] = jnp.ones((_L,), jnp.float32)

        pltpu.sync_copy(zbuf, cnt_sh.at[pl.ds(s * _SLICE, _SLICE)])
        plsc.subcore_barrier()

        tb = (c * _NS + s) * e_per
        pltpu.sync_copy(dst_h.at[pl.ds(tb, e_per)], dbuf)
        pltpu.sync_copy(typ_h.at[pl.ds(tb, e_per)], tbuf)

        @pl.loop(0, e_per // _L)
        def _(q):
            sl = pl.ds(q * _L, _L)
            tbuf[sl] = tbuf[sl] * _N + dbuf[sl]

        @pl.loop(0, n_ch)
        def _(j):
            for g in range(_K // _L):
                ibuf[0, pl.ds(g * _L, _L)] = tbuf[pl.ds(j * _K + g * _L, _L)]
            pltpu.sync_copy(ones, cnt_sh.at[ibuf.at[0]], add=True)

        plsc.subcore_barrier()
        pltpu.sync_copy(cnt_sh.at[pl.ds(s * _SLICE, _SLICE)], zbuf)
        pltpu.sync_copy(zbuf, out_h.at[pl.ds(c * _CNT_PAD + s * _SLICE, _SLICE)])

    return run(dst1, typ1)


def _sc_weights(src1, typ1, dst1, inv):
    e = src1.shape[0]
    e_per = e // (_NC * _NS)

    @functools.partial(
        pl.kernel,
        out_type=(jax.ShapeDtypeStruct((e,), jnp.int32),
                  jax.ShapeDtypeStruct((e,), jnp.float32)),
        mesh=plsc.VectorSubcoreMesh(**_MESH),
        scratch_types=[
            pltpu.VMEM((_CNT_PAD,), jnp.float32),
            pltpu.VMEM((e_per,), jnp.int32),
            pltpu.VMEM((e_per,), jnp.int32),
            pltpu.VMEM((e_per,), jnp.int32),
            pltpu.VMEM((e_per,), jnp.float32),
        ],
        compiler_params=pltpu.CompilerParams(needs_layout_passes=False),
    )
    def run(src_h, typ_h, dst_h, inv_h, g_out, w_out,
            inv_v, abuf, bbuf, cbuf, wbuf):
        c = lax.axis_index("c")
        s = lax.axis_index("s")
        pltpu.sync_copy(inv_h, inv_v)
        tb = (c * _NS + s) * e_per
        pltpu.sync_copy(src_h.at[pl.ds(tb, e_per)], abuf)
        pltpu.sync_copy(typ_h.at[pl.ds(tb, e_per)], bbuf)
        pltpu.sync_copy(dst_h.at[pl.ds(tb, e_per)], cbuf)

        @pl.loop(0, e_per // _L)
        def _(q):
            sl = pl.ds(q * _L, _L)
            t16 = bbuf[sl]
            abuf[sl] = t16 * _N + abuf[sl]
            d5 = t16 * _N + cbuf[sl]
            wbuf[sl] = plsc.load_gather(inv_v, [d5])

        pltpu.sync_copy(abuf, g_out.at[pl.ds(tb, e_per)])
        pltpu.sync_copy(wbuf, w_out.at[pl.ds(tb, e_per)])

    return run(src1, typ1, dst1, inv)


def _sc_edge(y, g1, dst1, w1):
    e_per = g1.shape[0] // (_NC * _NS)
    n_ch = e_per // _K
    zc = [_K] * (_RPT // _K) + ([_RPT % _K] if _RPT % _K else [])

    @functools.partial(
        pl.kernel,
        out_type=jax.ShapeDtypeStruct((_NC, _NPAD, _D), jnp.float32),
        mesh=plsc.VectorSubcoreMesh(**_MESH),
        scratch_types=[
            pltpu.VMEM((e_per,), jnp.int32),
            pltpu.VMEM((e_per,), jnp.int32),
            pltpu.VMEM((e_per,), jnp.float32),
            pltpu.VMEM((_K, _D), jnp.float32),
            pltpu.VMEM((1, _K), jnp.int32),
            pltpu.VMEM_SHARED((_NPAD, _D), jnp.float32),
            pltpu.SemaphoreType.DMA,
        ],
        compiler_params=pltpu.CompilerParams(needs_layout_passes=False),
    )
    def run(y_h, g_h, dst_h, w_h, out_h,
            abuf, cbuf, wbuf, rows, ibuf, acc_sh, gsem):
        c = lax.axis_index("c")
        s = lax.axis_index("s")

        @pl.loop(0, _K)
        def _(k):
            for i in range(_D // _L):
                rows[k, pl.ds(i * _L, _L)] = jnp.zeros((_L,), jnp.float32)

        off = 0
        for nch in zc:
            pltpu.sync_copy(rows.at[pl.ds(0, nch)],
                            acc_sh.at[pl.ds(s * _RPT + off, nch)])
            off += nch

        tb = (c * _NS + s) * e_per
        pltpu.sync_copy(g_h.at[pl.ds(tb, e_per)], abuf)
        pltpu.sync_copy(dst_h.at[pl.ds(tb, e_per)], cbuf)
        pltpu.sync_copy(w_h.at[pl.ds(tb, e_per)], wbuf)

        plsc.subcore_barrier()

        @pl.loop(0, n_ch)
        def _(j):
            pltpu.async_copy(y_h.at[abuf.at[pl.ds(j * _K, _K)]], rows,
                             gsem).wait()

            @pl.loop(0, _K // _L)
            def _(g):
                w16 = wbuf[pl.ds(j * _K + g * _L, _L)]
                ibuf[0, pl.ds(g * _L, _L)] = cbuf[pl.ds(j * _K + g * _L, _L)]
                for kk in range(_L):
                    w = w16[kk]
                    for i in range(_D // _L):
                        sl = pl.ds(i * _L, _L)
                        rows[g * _L + kk, sl] = rows[g * _L + kk, sl] * w

            pltpu.sync_copy(rows, acc_sh.at[ibuf.at[0]], add=True)

        plsc.subcore_barrier()

        off = 0
        for nch in zc:
            pltpu.sync_copy(acc_sh.at[pl.ds(s * _RPT + off, nch)],
                            rows.at[pl.ds(0, nch)])
            pltpu.sync_copy(rows.at[pl.ds(0, nch)],
                            out_h.at[c, pl.ds(s * _RPT + off, nch)])
            off += nch

    return run(y, g1, dst1, w1)


def kernel(des, tweet, num_prop, cat_prop, edge_index, edge_type,
           W_des, b_des, W_tw, b_tw, W_np, b_np, W_cp, b_cp,
           W_in, b_in, W_rel, W_root, b_rgcn, W_o1, b_o1, W_o2, b_o2):
    src1 = edge_index[0].astype(jnp.int32)
    dst1 = edge_index[1].astype(jnp.int32)
    typ1 = edge_type.astype(jnp.int32)

    npc = jnp.concatenate([num_prop, cat_prop], axis=1)
    npc = jnp.pad(npc, ((0, 0), (0, 128 - npc.shape[1])))
    nd = num_prop.shape[1]
    W_npc = jnp.zeros((128, 64), jnp.float32)
    W_npc = W_npc.at[:nd, :32].set(W_np).at[nd:nd + cat_prop.shape[1], 32:].set(W_cp)
    b_pre = jnp.concatenate([b_des, b_tw, b_np, b_cp]).reshape(1, 128)
    W_o2p = jnp.zeros((128, 128), jnp.float32).at[:, :W_o2.shape[1]].set(W_o2)
    b_o2p = jnp.zeros((128,), jnp.float32).at[:W_o2.shape[1]].set(b_o2).reshape(1, 128)

    cnts = _sc_count(dst1, typ1)
    inv = _inv(cnts.reshape(2, _CNT_PAD // 128, 128)).reshape(_CNT_PAD)
    g1, w1 = _sc_weights(src1, typ1, dst1, inv)

    x1 = _encoder(des, tweet, npc, W_des, W_tw, W_npc, b_pre,
                  W_in, b_in.reshape(1, 128))
    y1 = _relmm(x1, W_rel)
    p1 = _sc_edge(y1, g1, dst1, w1)
    x2 = _combine(x1, p1[0], p1[1], W_root, b_rgcn.reshape(1, 128))
    y2 = _relmm(x2, W_rel)
    p2 = _sc_edge(y2, g1, dst1, w1)
    out = _final(x2, p2[0], p2[1], W_root, b_rgcn.reshape(1, 128),
                 W_o1, b_o1.reshape(1, 128), W_o2p, b_o2p)
    return out[:, :W_o2.shape[1]]
